# baseline (speedup 1.0000x reference)
import functools

import jax
import jax.numpy as jnp
from jax import lax
from jax.experimental import pallas as pl
from jax.experimental.pallas import tpu as pltpu
from jax.experimental.pallas import tpu_sc as plsc

ALPHA = 200.0
B, N, M = 16, 512, 512
LANES = 16
CHUNK = 16
NCH = N // CHUNK
CVECS = M // LANES
UNROLL = 4


def _sm_body(s_hbm, nrow_hbm, ncol_hbm, out_hbm, buf, buf_t, zbuf,
             nrow_v, ncol_v):
    wid = lax.axis_index("s") * 2 + lax.axis_index("c")

    pltpu.sync_copy(nrow_hbm, nrow_v)
    pltpu.sync_copy(ncol_hbm, ncol_v)

    lanes = lax.iota(jnp.int32, LANES)
    zvec = jnp.zeros((LANES,), jnp.float32)

    def _zinit(j, carry):
        r = j // CVECS
        c = j % CVECS
        zbuf[r, pl.ds(c * LANES, LANES)] = zvec
        return carry

    lax.fori_loop(0, CHUNK * CVECS, _zinit, 0)

    nv = nrow_v[...]
    mv = ncol_v[...]

    for b in range(B):
        n = nv[b]
        m = mv[b]
        ch = lax.rem(wid + 2 * b, NCH)
        r0 = ch * CHUNK
        nblk = (m + LANES - 1) // LANES
        m4 = (m // UNROLL) * UNROLL

        @pl.when(r0 < n)
        def _compute():
            pltpu.sync_copy(s_hbm.at[b, pl.ds(r0, CHUNK), :], buf)
            rowv = (r0 + lanes) < n

            def _p1_col(c, acc):
                cf = jnp.full((LANES,), c, jnp.int32)
                x = plsc.load_gather(buf, [lanes, cf])
                buf_t[c, pl.ds(0, LANES)] = x
                return jnp.maximum(acc, x)

            def _p1(g, acc):
                for u in range(UNROLL):
                    acc = _p1_col(g * UNROLL + u, acc)
                return acc

            acc0 = jnp.full((LANES,), -3.0e38, jnp.float32)
            mvec = lax.fori_loop(0, m4 // UNROLL, _p1, acc0)
            rowmax = lax.fori_loop(m4, m, _p1_col, mvec)

            def _p2_col(c, acc):
                x = buf_t[c, pl.ds(0, LANES)]
                e = jnp.exp((x - rowmax) * ALPHA)
                buf_t[c, pl.ds(0, LANES)] = e
                return acc + e

            def _p2(g, acc):
                for u in range(UNROLL):
                    acc = _p2_col(g * UNROLL + u, acc)
                return acc

            svec = lax.fori_loop(0, m4 // UNROLL, _p2, zvec)
            denom = lax.fori_loop(m4, m, _p2_col, svec)
            scale = jnp.where(rowv, 1.0 / denom, 0.0)

            def _p3_col(c, carry):
                e = buf_t[c, pl.ds(0, LANES)]
                cf = jnp.full((LANES,), c, jnp.int32)
                plsc.store_scatter(buf, [lanes, cf], e * scale)
                return carry

            def _p3(g, carry):
                for u in range(UNROLL):
                    _p3_col(g * UNROLL + u, carry)
                return carry

            lax.fori_loop(0, m4 // UNROLL, _p3, 0)
            lax.fori_loop(m4, m, _p3_col, 0)

            def _ztcol(c, carry):
                cf = jnp.full((LANES,), c, jnp.int32)
                plsc.store_scatter(buf, [lanes, cf], zvec)
                return carry

            lax.fori_loop(m, nblk * LANES, _ztcol, 0)

            def _ztrow(r, carry):
                def _ztblk(cb, carry2):
                    buf[r, pl.ds(cb * LANES, LANES)] = zvec
                    return carry2

                lax.fori_loop(nblk, CVECS, _ztblk, 0)
                return carry

            lax.fori_loop(0, CHUNK, _ztrow, 0)
            pltpu.sync_copy(buf, out_hbm.at[b, pl.ds(r0, CHUNK), :])

        @pl.when(r0 >= n)
        def _zero():
            pltpu.sync_copy(zbuf, out_hbm.at[b, pl.ds(r0, CHUNK), :])


@jax.jit
def _sm_call(s, nrow_gt, ncol_gt):
    mesh = plsc.VectorSubcoreMesh(core_axis_name="c", subcore_axis_name="s")
    return pl.kernel(
        _sm_body,
        mesh=mesh,
        compiler_params=pltpu.CompilerParams(needs_layout_passes=False),
        out_type=jax.ShapeDtypeStruct((B, N, M), jnp.float32),
        scratch_types=[
            pltpu.VMEM((CHUNK, M), jnp.float32),
            pltpu.VMEM((M, CHUNK), jnp.float32),
            pltpu.VMEM((CHUNK, M), jnp.float32),
            pltpu.VMEM((LANES,), jnp.int32),
            pltpu.VMEM((LANES,), jnp.int32),
        ],
    )(s, nrow_gt, ncol_gt)


def kernel(s, nrow_gt, ncol_gt):
    return _sm_call(s, nrow_gt, ncol_gt)


# speedup vs baseline: 1.1294x; 1.1294x over previous
import functools

import jax
import jax.numpy as jnp
from jax import lax
from jax.experimental import pallas as pl
from jax.experimental.pallas import tpu as pltpu
from jax.experimental.pallas import tpu_sc as plsc

ALPHA = 200.0
B, N, M = 16, 512, 512
LANES = 16
CHUNK = 16
NCH = N // CHUNK
CVECS = M // LANES
UNROLL = 8


def _sm_body(s_hbm, nrow_hbm, ncol_hbm, out_hbm, buf, buf_t, zbuf,
             nrow_v, ncol_v):
    wid = lax.axis_index("s") * 2 + lax.axis_index("c")

    pltpu.sync_copy(nrow_hbm, nrow_v)
    pltpu.sync_copy(ncol_hbm, ncol_v)

    lanes = lax.iota(jnp.int32, LANES)
    zvec = jnp.zeros((LANES,), jnp.float32)

    def _zinit(j, carry):
        r = j // CVECS
        c = j % CVECS
        zbuf[r, pl.ds(c * LANES, LANES)] = zvec
        return carry

    lax.fori_loop(0, CHUNK * CVECS, _zinit, 0)

    nv = nrow_v[...]
    mv = ncol_v[...]

    def _batch(b, carry0):
        bf = jnp.full((LANES,), b, jnp.int32)
        n = nv.at[bf].get(mode="promise_in_bounds")[0]
        m = mv.at[bf].get(mode="promise_in_bounds")[0]
        ch = lax.rem(wid + 2 * b, NCH)
        r0 = ch * CHUNK
        nblk = (m + LANES - 1) // LANES
        m4 = (m // UNROLL) * UNROLL

        @pl.when(r0 < n)
        def _compute():
            pltpu.sync_copy(s_hbm.at[b, pl.ds(r0, CHUNK), :], buf)
            rowv = (r0 + lanes) < n

            def _p1_col(c, acc):
                cf = jnp.full((LANES,), c, jnp.int32)
                x = plsc.load_gather(buf, [lanes, cf])
                buf_t[c, pl.ds(0, LANES)] = x
                return jnp.maximum(acc, x)

            def _p1(g, accs):
                base = g * UNROLL
                return tuple(
                    _p1_col(base + u, accs[u]) for u in range(UNROLL))

            acc0 = jnp.full((LANES,), -3.0e38, jnp.float32)
            mvecs = lax.fori_loop(
                0, m4 // UNROLL, _p1, (acc0,) * UNROLL)
            mvec = functools.reduce(jnp.maximum, mvecs)
            rowmax = lax.fori_loop(m4, m, _p1_col, mvec)

            def _p2_col(c, acc):
                x = buf_t[c, pl.ds(0, LANES)]
                e = jnp.exp((x - rowmax) * ALPHA)
                buf_t[c, pl.ds(0, LANES)] = e
                return acc + e

            def _p2(g, accs):
                base = g * UNROLL
                return tuple(
                    _p2_col(base + u, accs[u]) for u in range(UNROLL))

            svecs = lax.fori_loop(0, m4 // UNROLL, _p2, (zvec,) * UNROLL)
            svec = functools.reduce(jnp.add, svecs)
            denom = lax.fori_loop(m4, m, _p2_col, svec)
            scale = jnp.where(rowv, 1.0 / denom, 0.0)

            def _p3_col(c, carry):
                e = buf_t[c, pl.ds(0, LANES)]
                cf = jnp.full((LANES,), c, jnp.int32)
                plsc.store_scatter(buf, [lanes, cf], e * scale)
                return carry

            def _p3(g, carry):
                base = g * UNROLL
                for u in range(UNROLL):
                    _p3_col(base + u, carry)
                return carry

            lax.fori_loop(0, m4 // UNROLL, _p3, 0)
            lax.fori_loop(m4, m, _p3_col, 0)

            def _ztcol(c, carry):
                cf = jnp.full((LANES,), c, jnp.int32)
                plsc.store_scatter(buf, [lanes, cf], zvec)
                return carry

            lax.fori_loop(m, nblk * LANES, _ztcol, 0)

            def _ztrow(r, carry):
                def _ztblk(cb, carry2):
                    buf[r, pl.ds(cb * LANES, LANES)] = zvec
                    return carry2

                lax.fori_loop(nblk, CVECS, _ztblk, 0)
                return carry

            lax.fori_loop(0, CHUNK, _ztrow, 0)
            pltpu.sync_copy(buf, out_hbm.at[b, pl.ds(r0, CHUNK), :])

        @pl.when(r0 >= n)
        def _zero():
            pltpu.sync_copy(zbuf, out_hbm.at[b, pl.ds(r0, CHUNK), :])

        return carry0

    lax.fori_loop(0, B, _batch, 0)


@jax.jit
def _sm_call(s, nrow_gt, ncol_gt):
    mesh = plsc.VectorSubcoreMesh(core_axis_name="c", subcore_axis_name="s")
    return pl.kernel(
        _sm_body,
        mesh=mesh,
        compiler_params=pltpu.CompilerParams(needs_layout_passes=False),
        out_type=jax.ShapeDtypeStruct((B, N, M), jnp.float32),
        scratch_types=[
            pltpu.VMEM((CHUNK, M), jnp.float32),
            pltpu.VMEM((M, CHUNK), jnp.float32),
            pltpu.VMEM((CHUNK, M), jnp.float32),
            pltpu.VMEM((LANES,), jnp.int32),
            pltpu.VMEM((LANES,), jnp.int32),
        ],
    )(s, nrow_gt, ncol_gt)


def kernel(s, nrow_gt, ncol_gt):
    return _sm_call(s, nrow_gt, ncol_gt)
